# baseline (speedup 1.0000x reference)
import functools

import jax
import jax.numpy as jnp
from jax import lax
from jax.experimental import pallas as pl
from jax.experimental.pallas import tpu as pltpu
from jax.experimental.pallas import tpu_sc as plsc

_LANES = 16


@functools.lru_cache(maxsize=None)
def _build(B, N, K, dtype_name):
    dtype = jnp.dtype(dtype_name)
    info = plsc.get_sparse_core_info()
    NC, NS = info.num_cores, info.num_subcores
    NW = NC * NS
    FLAT = B * N
    assert FLAT % NW == 0
    PER = FLAT // NW
    C = 16000
    NBB = 2
    NBUF = 2 * NBB
    assert PER % C == 0
    NCH = PER // C
    assert NCH % NBB == 0
    NG = NCH // NBB
    TI = B * K
    NP = NG
    assert TI % NP == 0
    PK = TI // NP
    assert PK % _LANES == 0
    PV = PK // _LANES
    UNROLL = 4
    assert PV % UNROLL == 0
    CAPN = 12800
    CROWS = CAPN // 128
    TCB = B // 128

    mesh = plsc.VectorSubcoreMesh(core_axis_name="c", subcore_axis_name="s")

    @functools.partial(
        pl.kernel,
        out_type=jax.ShapeDtypeStruct((FLAT,), dtype),
        mesh=mesh,
        scratch_types=(
            [pltpu.VMEM_SHARED((NS, C), dtype)] * NBUF
            + [pltpu.VMEM((PK,), jnp.int32)] * 2
            + [pltpu.VMEM((PK,), dtype)] * 2
            + [
                pltpu.VMEM((CROWS, 128), jnp.int32),
                pltpu.VMEM((CROWS, 128), dtype),
            ]
            + [pltpu.SemaphoreType.DMA] * NBUF
            + [pltpu.SemaphoreType.DMA] * NBUF
            + [pltpu.SemaphoreType.DMA] * 2
            + [pltpu.SemaphoreType.DMA] * 2
            + [pltpu.SemaphoreType.DMA]
        ),
        compiler_params=pltpu.CompilerParams(
            needs_layout_passes=False,
            disable_bounds_checks=True,
        ),
    )
    def run(x_hbm, idx_hbm, src_hbm, out_hbm, *scratch):
        bufs = scratch[:NBUF]
        ibufs = scratch[NBUF:NBUF + 2]
        sbufs = scratch[NBUF + 2:NBUF + 4]
        wlp, wlv = scratch[NBUF + 4], scratch[NBUF + 5]
        rsem = scratch[NBUF + 6:2 * NBUF + 6]
        wsem = scratch[2 * NBUF + 6:3 * NBUF + 6]
        isem = scratch[3 * NBUF + 6:3 * NBUF + 8]
        ssem = scratch[3 * NBUF + 8:3 * NBUF + 10]
        csem = scratch[3 * NBUF + 10]
        sid = lax.axis_index("s")
        wid = sid * NC + lax.axis_index("c")
        lanes = lax.iota(jnp.int32, _LANES)
        F0 = wid * PER

        def start_read(b, t):
            pltpu.async_copy(x_hbm.at[pl.ds(F0 + t * C, C)], bufs[b].at[sid],
                             rsem[b])

        def wait_read(b):
            pltpu.make_async_copy(x_hbm.at[pl.ds(0, C)], bufs[b].at[sid],
                                  rsem[b]).wait()

        def start_write(b, t):
            pltpu.async_copy(bufs[b].at[sid], out_hbm.at[pl.ds(F0 + t * C, C)],
                             wsem[b])

        def wait_write(b):
            pltpu.make_async_copy(bufs[b].at[sid], out_hbm.at[pl.ds(0, C)],
                                  wsem[b]).wait()

        def start_piece(p, par):
            pltpu.async_copy(idx_hbm.at[pl.ds(p * PK, PK)], ibufs[par],
                             isem[par])
            pltpu.async_copy(src_hbm.at[pl.ds(p * PK, PK)], sbufs[par],
                             ssem[par])

        def wait_piece(par):
            pltpu.make_async_copy(idx_hbm.at[pl.ds(0, PK)], ibufs[par],
                                  isem[par]).wait()
            pltpu.make_async_copy(src_hbm.at[pl.ds(0, PK)], sbufs[par],
                                  ssem[par]).wait()

        def scan_piece(p, par, carry):
            ib, sb = ibufs[par], sbufs[par]
            pbase = p * PK

            def one(off, carry):
                cnt, lp, lv, lm = carry
                j = ib[pl.ds(off, _LANES)]
                s = sb[pl.ds(off, _LANES)]
                pos = (pbase + off) + lanes
                i = pos // K
                phys = ((j >> 3) * (TCB * 1024) + ((i >> 7) << 10)
                        + ((j & 7) << 7) + (i & 127))
                m = (phys >= F0) & (phys < F0 + PER)
                mi = m.astype(jnp.int32)
                pc = plsc.cumsum(mi)
                tot = jnp.sum(mi)
                slot = cnt + pc - 1
                keep = m & (slot < CAPN)
                slot = jnp.where(keep, slot, 0)
                plsc.store_scatter(wlp, [slot >> 7, slot & 127], phys,
                                   mask=keep)
                plsc.store_scatter(wlv, [slot >> 7, slot & 127], s,
                                   mask=keep)
                some = tot > 0
                return (cnt + tot,
                        jnp.where(some, phys, lp),
                        jnp.where(some, s, lv),
                        jnp.where(some, mi, lm))

            def vbody(u, carry):
                for q in range(UNROLL):
                    carry = one(u * (UNROLL * _LANES) + q * _LANES, carry)
                return carry

            return lax.fori_loop(0, PV // UNROLL, vbody, carry)

        for k in range(NBB):
            start_read(k, k)
        start_piece(0, 0)

        def group(g, par, carry, first=False, last=False):
            A = range(par * NBB, par * NBB + NBB)
            Bk = range((1 - par) * NBB, (1 - par) * NBB + NBB)
            for k, b in enumerate(A):
                wait_read(b)
                start_write(b, g * NBB + k)
            wait_piece(par)
            carry = scan_piece(g, par, carry)
            if not last:
                start_piece(g + 1, 1 - par)
                for k, b in enumerate(Bk):
                    if not first:
                        wait_write(b)
                    start_read(b, (g + 1) * NBB + k)
            return carry

        carry0 = (jnp.int32(0), jnp.zeros((_LANES,), jnp.int32),
                  jnp.zeros((_LANES,), dtype), jnp.zeros((_LANES,), jnp.int32))

        assert NG % 2 == 0
        carry = group(0, 0, carry0, first=True)

        def body(h, carry):
            g = h * 2 + 1
            carry = group(g, 1, carry)
            carry = group(g + 1, 0, carry)
            return carry

        carry = lax.fori_loop(0, NG // 2 - 1, body, carry)
        cnt, lp_v, lv_v, lm_v = group(NG - 1, 1, carry, last=True)
        for b in range(NBUF):
            wait_write(b)

        cl = jnp.minimum(cnt, CAPN)
        pcl = plsc.cumsum(lm_v)
        is_last = (lm_v > 0) & (pcl == jnp.sum(lm_v))
        lp = jnp.sum(jnp.where(is_last, lp_v, 0))
        lv = jnp.sum(jnp.where(is_last, lv_v, jnp.zeros((), dtype)))
        rnd = ((cl + 127) // 128) * 128
        for kpad in range(128 // _LANES):
            slotv = cl + kpad * _LANES + lanes
            mk = slotv < rnd
            slotv = jnp.where(mk, slotv, 0)
            plsc.store_scatter(wlp, [slotv >> 7, slotv & 127],
                               jnp.full((_LANES,), 1, jnp.int32) * lp,
                               mask=mk)
            plsc.store_scatter(wlv, [slotv >> 7, slotv & 127],
                               jnp.full((_LANES,), 1, dtype) * lv,
                               mask=mk)

        npieces = rnd // 128

        def fire(k, c):
            pltpu.async_copy(wlv.at[k], out_hbm.at[wlp.at[k]], csem)
            return c

        def drain(k, c):
            pltpu.make_async_copy(wlv.at[0], out_hbm.at[wlp.at[0]],
                                  csem).wait()
            return c

        lax.fori_loop(0, npieces, fire, 0)
        lax.fori_loop(0, npieces, drain, 0)

    return run


def kernel(x, indices, src):
    B, N = x.shape
    K = indices.shape[1]
    run = _build(B, N, K, jnp.dtype(x.dtype).name)
    xflat = (x.T.reshape(N // 8, 8, B // 128, 128)
             .transpose(0, 2, 1, 3).reshape(-1))
    outflat = run(
        xflat,
        indices.astype(jnp.int32).reshape(-1),
        src.astype(x.dtype).reshape(-1),
    )
    return (outflat.reshape(N // 8, B // 128, 8, 128)
            .transpose(0, 2, 1, 3).reshape(N, B).T)


# speedup vs baseline: 1.0838x; 1.0838x over previous
import functools

import jax
import jax.numpy as jnp
from jax import lax
from jax.experimental import pallas as pl
from jax.experimental.pallas import tpu as pltpu
from jax.experimental.pallas import tpu_sc as plsc

_LANES = 16


@functools.lru_cache(maxsize=None)
def _build(B, N, K, dtype_name):
    dtype = jnp.dtype(dtype_name)
    info = plsc.get_sparse_core_info()
    NC, NS = info.num_cores, info.num_subcores
    NW = NC * NS
    FLAT = B * N
    assert FLAT % NW == 0
    PER = FLAT // NW
    C = 16000
    NBUF = 5
    assert PER % C == 0
    NCH = PER // C
    assert NCH % NBUF == 0
    NG = NCH // NBUF
    TI = B * K
    NP = NG
    assert TI % NP == 0
    PK = TI // NP
    assert PK % _LANES == 0
    PV = PK // _LANES
    UNROLL = 4
    assert PV % UNROLL == 0
    CAPN = 12800
    CROWS = CAPN // 128
    TCB = B // 128

    mesh = plsc.VectorSubcoreMesh(core_axis_name="c", subcore_axis_name="s")

    @functools.partial(
        pl.kernel,
        out_type=jax.ShapeDtypeStruct((FLAT,), dtype),
        mesh=mesh,
        scratch_types=(
            [pltpu.VMEM_SHARED((NS, C), dtype)] * NBUF
            + [pltpu.VMEM((PK,), jnp.int32)] * 2
            + [pltpu.VMEM((PK,), dtype)] * 2
            + [
                pltpu.VMEM((CROWS, 128), jnp.int32),
                pltpu.VMEM((CROWS, 128), dtype),
            ]
            + [pltpu.SemaphoreType.DMA] * NBUF
            + [pltpu.SemaphoreType.DMA] * NBUF
            + [pltpu.SemaphoreType.DMA] * 2
            + [pltpu.SemaphoreType.DMA] * 2
            + [pltpu.SemaphoreType.DMA]
        ),
        compiler_params=pltpu.CompilerParams(
            needs_layout_passes=False,
            disable_bounds_checks=True,
        ),
    )
    def run(x_hbm, idx_hbm, src_hbm, out_hbm, *scratch):
        bufs = scratch[:NBUF]
        ibufs = scratch[NBUF:NBUF + 2]
        sbufs = scratch[NBUF + 2:NBUF + 4]
        wlp, wlv = scratch[NBUF + 4], scratch[NBUF + 5]
        rsem = scratch[NBUF + 6:2 * NBUF + 6]
        wsem = scratch[2 * NBUF + 6:3 * NBUF + 6]
        isem = scratch[3 * NBUF + 6:3 * NBUF + 8]
        ssem = scratch[3 * NBUF + 8:3 * NBUF + 10]
        csem = scratch[3 * NBUF + 10]
        sid = lax.axis_index("s")
        wid = sid * NC + lax.axis_index("c")
        lanes = lax.iota(jnp.int32, _LANES)
        F0 = wid * PER

        def start_read(b, t):
            pltpu.async_copy(x_hbm.at[pl.ds(F0 + t * C, C)], bufs[b].at[sid],
                             rsem[b])

        def wait_read(b):
            pltpu.make_async_copy(x_hbm.at[pl.ds(0, C)], bufs[b].at[sid],
                                  rsem[b]).wait()

        def start_write(b, t):
            pltpu.async_copy(bufs[b].at[sid], out_hbm.at[pl.ds(F0 + t * C, C)],
                             wsem[b])

        def wait_write(b):
            pltpu.make_async_copy(bufs[b].at[sid], out_hbm.at[pl.ds(0, C)],
                                  wsem[b]).wait()

        def start_piece(p, par):
            pltpu.async_copy(idx_hbm.at[pl.ds(p * PK, PK)], ibufs[par],
                             isem[par])
            pltpu.async_copy(src_hbm.at[pl.ds(p * PK, PK)], sbufs[par],
                             ssem[par])

        def wait_piece(par):
            pltpu.make_async_copy(idx_hbm.at[pl.ds(0, PK)], ibufs[par],
                                  isem[par]).wait()
            pltpu.make_async_copy(src_hbm.at[pl.ds(0, PK)], sbufs[par],
                                  ssem[par]).wait()

        def scan_piece(p, par, carry):
            ib, sb = ibufs[par], sbufs[par]
            pbase = p * PK

            def one(off, carry):
                cnt, lp, lv, lm = carry
                j = ib[pl.ds(off, _LANES)]
                s = sb[pl.ds(off, _LANES)]
                pos = (pbase + off) + lanes
                i = pos // K
                phys = ((j >> 3) * (TCB * 1024) + ((i >> 7) << 10)
                        + ((j & 7) << 7) + (i & 127))
                m = (phys >= F0) & (phys < F0 + PER)
                mi = m.astype(jnp.int32)
                pc = plsc.cumsum(mi)
                tot = jnp.sum(mi)
                slot = cnt + pc - 1
                keep = m & (slot < CAPN)
                slot = jnp.where(keep, slot, 0)
                plsc.store_scatter(wlp, [slot >> 7, slot & 127], phys,
                                   mask=keep)
                plsc.store_scatter(wlv, [slot >> 7, slot & 127], s,
                                   mask=keep)
                some = tot > 0
                return (cnt + tot,
                        jnp.where(some, phys, lp),
                        jnp.where(some, s, lv),
                        jnp.where(some, mi, lm))

            def vbody(u, carry):
                for q in range(UNROLL):
                    carry = one(u * (UNROLL * _LANES) + q * _LANES, carry)
                return carry

            return lax.fori_loop(0, PV // UNROLL, vbody, carry)

        for b in range(NBUF):
            start_read(b, b)
        start_piece(0, 0)

        def group(g, par, carry, prefetch):
            for b in range(NBUF):
                wait_read(b)
                start_write(b, g * NBUF + b)
            wait_piece(par)
            carry = scan_piece(g, par, carry)
            if prefetch:
                start_piece(g + 1, 1 - par)
                for b in range(NBUF):
                    wait_write(b)
                    start_read(b, (g + 1) * NBUF + b)
            return carry

        carry0 = (jnp.int32(0), jnp.zeros((_LANES,), jnp.int32),
                  jnp.zeros((_LANES,), dtype), jnp.zeros((_LANES,), jnp.int32))

        assert NG % 2 == 0

        def body(h, carry):
            g = h * 2
            carry = group(g, 0, carry, prefetch=True)
            carry = group(g + 1, 1, carry, prefetch=True)
            return carry

        carry = lax.fori_loop(0, NG // 2 - 1, body, carry0)
        carry = group(NG - 2, 0, carry, prefetch=True)
        cnt, lp_v, lv_v, lm_v = group(NG - 1, 1, carry, prefetch=False)
        for b in range(NBUF):
            wait_write(b)

        cl = jnp.minimum(cnt, CAPN)
        pcl = plsc.cumsum(lm_v)
        is_last = (lm_v > 0) & (pcl == jnp.sum(lm_v))
        lp = jnp.sum(jnp.where(is_last, lp_v, 0))
        lv = jnp.sum(jnp.where(is_last, lv_v, jnp.zeros((), dtype)))
        rnd = ((cl + 127) // 128) * 128
        for kpad in range(128 // _LANES):
            slotv = cl + kpad * _LANES + lanes
            mk = slotv < rnd
            slotv = jnp.where(mk, slotv, 0)
            plsc.store_scatter(wlp, [slotv >> 7, slotv & 127],
                               jnp.full((_LANES,), 1, jnp.int32) * lp,
                               mask=mk)
            plsc.store_scatter(wlv, [slotv >> 7, slotv & 127],
                               jnp.full((_LANES,), 1, dtype) * lv,
                               mask=mk)

        npieces = rnd // 128

        def fire(k, c):
            pltpu.async_copy(wlv.at[k], out_hbm.at[wlp.at[k]], csem)
            return c

        def drain(k, c):
            pltpu.make_async_copy(wlv.at[0], out_hbm.at[wlp.at[0]],
                                  csem).wait()
            return c

        lax.fori_loop(0, npieces, fire, 0)
        lax.fori_loop(0, npieces, drain, 0)

    return run


def kernel(x, indices, src):
    B, N = x.shape
    K = indices.shape[1]
    run = _build(B, N, K, jnp.dtype(x.dtype).name)
    xflat = (x.T.reshape(N // 8, 8, B // 128, 128)
             .transpose(0, 2, 1, 3).reshape(-1))
    outflat = run(
        xflat,
        indices.astype(jnp.int32).reshape(-1),
        src.astype(x.dtype).reshape(-1),
    )
    return (outflat.reshape(N // 8, B // 128, 8, 128)
            .transpose(0, 2, 1, 3).reshape(N, B).T)
